# baseline (speedup 1.0000x reference)
import functools

import jax
import jax.numpy as jnp
from jax import lax
from jax.experimental import pallas as pl
from jax.experimental.pallas import tpu as pltpu
from jax.experimental.pallas import tpu_sc as plsc


def kernel(x, edge_index, expander_edges, virt_table):
    num_node = x.shape[0]
    num_virt = virt_table.shape[0]
    n_edges = edge_index.shape[1]
    edtype = edge_index.dtype

    seg = num_node
    n_seg = 2 * num_virt
    ve_row = n_seg * num_node
    chunk = n_edges // 8

    mesh = plsc.VectorSubcoreMesh(core_axis_name="c", subcore_axis_name="s")

    @functools.partial(
        pl.kernel,
        out_type=(
            jax.ShapeDtypeStruct((2 * 2 * n_edges,), edtype),
            jax.ShapeDtypeStruct(virt_table.shape, virt_table.dtype),
            jax.ShapeDtypeStruct((2 * ve_row,), edtype),
        ),
        mesh=mesh,
        scratch_types=[
            pltpu.VMEM((seg,), jnp.int32),
            pltpu.VMEM((chunk,), jnp.int32),
            pltpu.SemaphoreType.DMA,
        ],
    )
    def sc_kernel(edge_hbm, exp_hbm, virt_hbm, out_e, out_v, out_ve,
                  buf, ebuf, sem):
        wid = lax.axis_index("s") * 2 + lax.axis_index("c")

        h = wid // 16
        r2 = (wid // 8) % 2
        c = wid % 8
        src = pl.ds(r2 * n_edges + c * chunk, chunk)
        dst = pl.ds(r2 * 2 * n_edges + h * n_edges + c * chunk, chunk)

        @pl.when(h == 0)
        def _():
            pltpu.make_async_copy(edge_hbm.at[src], ebuf, sem).start()

        @pl.when(h == 1)
        def _():
            pltpu.make_async_copy(exp_hbm.at[src], ebuf, sem).start()

        s = wid // 2
        r = wid % 2
        is_ar = (s % 2) == r
        iota = lax.iota(jnp.int32, 16)
        cvec = jnp.full((16,), num_node, jnp.int32) + (s // 2)

        def body(i, carry):
            v = jnp.where(is_ar, iota + i * 16, cvec)
            buf[pl.ds(i * 16, 16)] = v
            return carry

        lax.fori_loop(0, seg // 16, body, 0)
        pltpu.sync_copy(buf, out_ve.at[pl.ds(r * ve_row + s * seg, seg)])

        pltpu.make_async_copy(edge_hbm.at[src], ebuf, sem).wait()
        pltpu.sync_copy(ebuf, out_e.at[dst])

        @pl.when(wid == 0)
        def _():
            pltpu.sync_copy(virt_hbm, out_v)

    out_e, out_v, out_ve = sc_kernel(
        edge_index.reshape(-1), expander_edges.reshape(-1), virt_table)
    return (out_e.reshape(2, 2 * n_edges), out_v, out_ve.reshape(2, ve_row))


# speedup vs baseline: 1.0059x; 1.0059x over previous
import functools

import jax
import jax.numpy as jnp
from jax import lax
from jax.experimental import pallas as pl
from jax.experimental.pallas import tpu as pltpu
from jax.experimental.pallas import tpu_sc as plsc


def kernel(x, edge_index, expander_edges, virt_table):
    num_node = x.shape[0]
    num_virt = virt_table.shape[0]
    n_edges = edge_index.shape[1]
    edtype = edge_index.dtype

    seg = num_node
    n_seg = 2 * num_virt
    ve_row = n_seg * num_node
    chunk = n_edges // 8

    mesh = plsc.VectorSubcoreMesh(core_axis_name="c", subcore_axis_name="s")

    @functools.partial(
        pl.kernel,
        out_type=(
            jax.ShapeDtypeStruct((2 * 2 * n_edges,), edtype),
            jax.ShapeDtypeStruct(virt_table.shape, virt_table.dtype),
            jax.ShapeDtypeStruct((2 * ve_row,), edtype),
        ),
        mesh=mesh,
        scratch_types=[
            pltpu.VMEM((seg,), jnp.int32),
            pltpu.VMEM((chunk,), jnp.int32),
            pltpu.SemaphoreType.DMA,
            pltpu.SemaphoreType.DMA,
            pltpu.SemaphoreType.DMA,
        ],
    )
    def sc_kernel(edge_hbm, exp_hbm, virt_hbm, out_e, out_v, out_ve,
                  buf, ebuf, sem, wsem_v, wsem_e):
        wid = lax.axis_index("s") * 2 + lax.axis_index("c")

        h = wid // 16
        r2 = (wid // 8) % 2
        c = wid % 8
        src = pl.ds(r2 * n_edges + c * chunk, chunk)
        dst = pl.ds(r2 * 2 * n_edges + h * n_edges + c * chunk, chunk)

        @pl.when(h == 0)
        def _():
            pltpu.make_async_copy(edge_hbm.at[src], ebuf, sem).start()

        @pl.when(h == 1)
        def _():
            pltpu.make_async_copy(exp_hbm.at[src], ebuf, sem).start()

        s = wid // 2
        r = wid % 2
        is_ar = (s % 2) == r
        iota = lax.iota(jnp.int32, 16)
        cvec = jnp.full((16,), num_node, jnp.int32) + (s // 2)
        v0 = jnp.where(is_ar, iota, cvec)
        inc = jnp.where(is_ar, jnp.full((16,), 16, jnp.int32),
                        jnp.zeros((16,), jnp.int32))

        U = 5

        def body(i, v):
            base = i * (16 * U)
            for j in range(U):
                buf[pl.ds(base + j * 16, 16)] = v
                v = v + inc
            return v

        lax.fori_loop(0, seg // (16 * U), body, v0)
        ve_dst = out_ve.at[pl.ds(r * ve_row + s * seg, seg)]
        pltpu.make_async_copy(buf, ve_dst, wsem_v).start()

        pltpu.make_async_copy(edge_hbm.at[src], ebuf, sem).wait()
        pltpu.make_async_copy(ebuf, out_e.at[dst], wsem_e).start()

        @pl.when(wid == 0)
        def _():
            pltpu.sync_copy(virt_hbm, out_v)

        pltpu.make_async_copy(buf, ve_dst, wsem_v).wait()
        pltpu.make_async_copy(ebuf, out_e.at[dst], wsem_e).wait()

    out_e, out_v, out_ve = sc_kernel(
        edge_index.reshape(-1), expander_edges.reshape(-1), virt_table)
    return (out_e.reshape(2, 2 * n_edges), out_v, out_ve.reshape(2, ve_row))
